# baseline (speedup 1.0000x reference)
import functools

import jax
import jax.numpy as jnp
from jax import lax
from jax.experimental import pallas as pl
from jax.experimental.pallas import tpu as pltpu
from jax.experimental.pallas import tpu_sc as plsc

_NUM_CORES = 2
_NUM_SUBCORES = 16
_NUM_WORKERS = _NUM_CORES * _NUM_SUBCORES

_V_BLK = 1024
_NBUF = 8


@functools.cache
def _make_sc_gather(V, D, B):
    b_per_w = B // _NUM_WORKERS
    mesh = plsc.VectorSubcoreMesh(core_axis_name="c", subcore_axis_name="s")

    @functools.partial(
        pl.kernel,
        mesh=mesh,
        out_type=jax.ShapeDtypeStruct((B, D), jnp.float32),
        scratch_types=[
            pltpu.VMEM((b_per_w,), jnp.int32),
            pltpu.VMEM((b_per_w, D), jnp.float32),
            pltpu.SemaphoreType.DMA,
        ],
        compiler_params=pltpu.CompilerParams(use_tc_tiling_on_sc=False),
    )
    def sc_gather(table_hbm, idx_hbm, out_hbm, idx_v, rows_v, sem):
        wid = lax.axis_index("s") * _NUM_CORES + lax.axis_index("c")
        base = wid * b_per_w
        pltpu.sync_copy(idx_hbm.at[pl.ds(base, b_per_w)], idx_v)
        pltpu.async_copy(table_hbm.at[idx_v], rows_v, sem).wait()
        pltpu.sync_copy(rows_v, out_hbm.at[pl.ds(base, b_per_w)])

    return sc_gather


@functools.cache
def _make_tc_matmul(V, D, B):
    v_aligned = (V // 128) * 128
    nblk = (v_aligned + _V_BLK - 1) // _V_BLK
    tail = v_aligned - (nblk - 1) * _V_BLK

    def body(emb_ref, w_ref, b_ref, out_ref, acc, sems):
        i = pl.program_id(0)
        slot = lax.rem(i, _NBUF)

        @pl.when(i >= _NBUF)
        def _wait_prev():
            pltpu.make_async_copy(
                acc.at[slot],
                out_ref.at[:, pl.ds(0, _V_BLK)],
                sems.at[slot],
            ).wait()

        acc[slot] = (
            lax.dot_general(
                emb_ref[...],
                w_ref[...],
                (((1,), (1,)), ((), ())),
                preferred_element_type=jnp.float32,
            )
            + b_ref[...]
        )

        @pl.when(i < nblk - 1)
        def _issue_full():
            pltpu.make_async_copy(
                acc.at[slot],
                out_ref.at[:, pl.ds(i * _V_BLK, _V_BLK)],
                sems.at[slot],
            ).start()

        @pl.when(i == nblk - 1)
        def _issue_tail_and_drain():
            pltpu.make_async_copy(
                acc.at[slot, :, pl.ds(0, tail)],
                out_ref.at[:, pl.ds((nblk - 1) * _V_BLK, tail)],
                sems.at[slot],
            ).start()
            for j in range(max(0, nblk - _NBUF), nblk):
                w_cols = tail if j == nblk - 1 else _V_BLK
                pltpu.make_async_copy(
                    acc.at[j % _NBUF, :, pl.ds(0, w_cols)],
                    out_ref.at[:, pl.ds(0, w_cols)],
                    sems.at[j % _NBUF],
                ).wait()

    return pl.pallas_call(
        body,
        grid=(nblk,),
        in_specs=[
            pl.BlockSpec((B, D), lambda i: (0, 0)),
            pl.BlockSpec((_V_BLK, D), lambda i: (i, 0)),
            pl.BlockSpec((1, _V_BLK), lambda i: (0, i)),
        ],
        out_specs=pl.BlockSpec(memory_space=pltpu.HBM),
        out_shape=jax.ShapeDtypeStruct((B, V), jnp.float32),
        scratch_shapes=[
            pltpu.VMEM((_NBUF, B, _V_BLK), jnp.float32),
            pltpu.SemaphoreType.DMA((_NBUF,)),
        ],
    )


@functools.cache
def _make_tc_tail(V, D, B):
    blk_idx = V // 128

    def body(big_ref, emb_ref, w_ref, b_ref, out_ref):
        del big_ref
        out_ref[...] = (
            lax.dot_general(
                emb_ref[...],
                w_ref[...],
                (((1,), (1,)), ((), ())),
                preferred_element_type=jnp.float32,
            )
            + b_ref[...]
        )

    return pl.pallas_call(
        body,
        grid=(1,),
        in_specs=[
            pl.BlockSpec(memory_space=pltpu.HBM),
            pl.BlockSpec((B, D), lambda i: (0, 0)),
            pl.BlockSpec((128, D), lambda i: (0, 0)),
            pl.BlockSpec((1, 128), lambda i: (0, 0)),
        ],
        out_specs=pl.BlockSpec((B, 128), lambda i: (0, blk_idx)),
        out_shape=jax.ShapeDtypeStruct((B, V), jnp.float32),
        input_output_aliases={0: 0},
    )


def kernel(x, tok_embeddings, fc_w, fc_b):
    V, D = tok_embeddings.shape
    B = x.shape[0]
    emb = _make_sc_gather(V, D, B)(tok_embeddings, x.astype(jnp.int32))
    out = _make_tc_matmul(V, D, B)(emb, fc_w, fc_b.reshape(1, V))
    v_aligned = (V // 128) * 128
    if v_aligned < V:
        rem = V - v_aligned
        w_tail = jnp.pad(fc_w[v_aligned:], ((0, 128 - rem), (0, 0)))
        b_tail = jnp.pad(fc_b[v_aligned:], (0, 128 - rem)).reshape(1, 128)
        out = _make_tc_tail(V, D, B)(out, emb, w_tail, b_tail)
    return out


# speedup vs baseline: 1.0895x; 1.0895x over previous
import functools

import jax
import jax.numpy as jnp
from jax import lax
from jax.experimental import pallas as pl
from jax.experimental.pallas import tpu as pltpu
from jax.experimental.pallas import tpu_sc as plsc

_NUM_CORES = 2
_NUM_SUBCORES = 16
_NUM_WORKERS = _NUM_CORES * _NUM_SUBCORES

_B_BLK = 64


@functools.cache
def _make_sc_gather(V, D, B):
    b_per_w = B // _NUM_WORKERS
    mesh = plsc.VectorSubcoreMesh(core_axis_name="c", subcore_axis_name="s")

    @functools.partial(
        pl.kernel,
        mesh=mesh,
        out_type=jax.ShapeDtypeStruct((B, D), jnp.float32),
        scratch_types=[
            pltpu.VMEM((b_per_w,), jnp.int32),
            pltpu.VMEM((b_per_w, D), jnp.float32),
            pltpu.SemaphoreType.DMA,
        ],
        compiler_params=pltpu.CompilerParams(use_tc_tiling_on_sc=False),
    )
    def sc_gather(table_hbm, idx_hbm, out_hbm, idx_v, rows_v, sem):
        wid = lax.axis_index("s") * _NUM_CORES + lax.axis_index("c")
        base = wid * b_per_w
        pltpu.sync_copy(idx_hbm.at[pl.ds(base, b_per_w)], idx_v)
        pltpu.async_copy(table_hbm.at[idx_v], rows_v, sem).wait()
        pltpu.sync_copy(rows_v, out_hbm.at[pl.ds(base, b_per_w)])

    return sc_gather


def _tc_matmul_body(emb_ref, wt_ref, b_ref, out_ref):
    out_ref[...] = (
        lax.dot_general(
            emb_ref[...],
            wt_ref[...],
            (((1,), (0,)), ((), ())),
            preferred_element_type=jnp.float32,
        )
        + b_ref[...]
    )


@functools.cache
def _make_tc_matmul(V, D, B):
    nsteps = B // _B_BLK
    return pl.pallas_call(
        _tc_matmul_body,
        grid=(nsteps,),
        in_specs=[
            pl.BlockSpec((_B_BLK, D), lambda i: (i, 0)),
            pl.BlockSpec((D, V), lambda i: (0, 0)),
            pl.BlockSpec((1, V), lambda i: (0, 0)),
        ],
        out_specs=pl.BlockSpec((_B_BLK, V), lambda i: (i, 0)),
        out_shape=jax.ShapeDtypeStruct((B, V), jnp.float32),
        compiler_params=pltpu.CompilerParams(
            vmem_limit_bytes=110 * 1024 * 1024,
        ),
    )


def kernel(x, tok_embeddings, fc_w, fc_b):
    V, D = tok_embeddings.shape
    B = x.shape[0]
    emb = _make_sc_gather(V, D, B)(tok_embeddings, x.astype(jnp.int32))
    return _make_tc_matmul(V, D, B)(emb, fc_w.T, fc_b.reshape(1, V))
